# baseline (device time: 33705 ns/iter reference)
import jax
import jax.numpy as jnp
from jax import lax
from jax.experimental import pallas as pl
from jax.experimental.pallas import tpu as pltpu

N_DEV = 4
B, Sq, Skv, Dh = 2, 512, 512, 64
HQ_LOCAL = 8
D_MODEL = 768
HALF = D_MODEL // 2
WINDOW = 128
SCALE = 0.125
TILE = 128
N_TILES = Sq // TILE
C = B * N_TILES
CHUNKS = [(b, i) for b in range(B) for i in range(N_TILES)]
R1_DEPTH = 2
R2_DEPTH = 5


def _band(i):
    return max(0, TILE * (i - 1)), min(Skv, TILE * (i + 2))


def _body(x_ref, wq_ref, k_ref, v_ref, wo_ref, out_ref,
          qproj_ref, recvbuf, send_sems, recv_sems):
    my = lax.axis_index("i")
    p1 = my ^ 1
    p2 = 3 - my
    partners = ((p1, p2), (p2, p1))

    barrier_sem = pltpu.get_barrier_semaphore()
    for p in (p1, p2):
        pl.semaphore_signal(
            barrier_sem, inc=1,
            device_id=(p,), device_id_type=pl.DeviceIdType.MESH,
        )
    pl.semaphore_wait(barrier_sem, 2)

    for b in range(B):
        qproj_ref[b] = jnp.dot(
            x_ref[b], wq_ref[...], preferred_element_type=jnp.float32
        ).astype(jnp.bfloat16)

    def make_rdma(r, c, t):
        b, i = CHUNKS[c]
        r0 = i * TILE
        cs = slice(t * HALF, (t + 1) * HALF)
        return pltpu.make_async_remote_copy(
            src_ref=out_ref.at[b, r0:r0 + TILE, cs],
            dst_ref=recvbuf.at[r, c, :, cs],
            send_sem=send_sems.at[r, c, t],
            recv_sem=recv_sems.at[r, c, t],
            device_id=(partners[r][t],),
            device_id_type=pl.DeviceIdType.MESH,
        )

    def compute_chunk(c):
        b, i = CHUNKS[c]
        r0 = i * TILE
        q_b = qproj_ref[b, r0:r0 + TILE, :]
        j0, j1 = _band(i)
        colw = j1 - j0
        ri = lax.broadcasted_iota(jnp.int32, (TILE, colw), 0) + r0
        ci = lax.broadcasted_iota(jnp.int32, (TILE, colw), 1) + j0
        mask = jnp.abs(ri - ci) <= WINDOW
        ctx_cols = []
        for h in range(HQ_LOCAL):
            sl = slice(h * Dh, (h + 1) * Dh)
            q = q_b[:, sl]
            k = k_ref[b, j0:j1, sl]
            s = lax.dot_general(
                q, k, (((1,), (1,)), ((), ())),
                preferred_element_type=jnp.float32,
            ).astype(jnp.bfloat16)
            w = jnp.exp2(jnp.where(mask, s, jnp.bfloat16(-1e9)))
            denom = 1.0 / jnp.sum(w.astype(jnp.float32), axis=1,
                                  keepdims=True)
            v = v_ref[b, j0:j1, sl]
            ctx_h = lax.dot_general(
                w, v, (((1,), (0,)), ((), ())),
                preferred_element_type=jnp.float32,
            ) * denom
            ctx_cols.append(ctx_h.astype(jnp.bfloat16))
        ctx = jnp.concatenate(ctx_cols, axis=1)
        part = jnp.dot(
            ctx, wo_ref[...], preferred_element_type=jnp.float32
        )
        out_ref[b, r0:r0 + TILE, :] = part.astype(jnp.bfloat16)

    r1 = [[None, None] for _ in range(C)]
    r2 = [[None, None] for _ in range(C)]

    def finish_r1(c):
        b, i = CHUNKS[c]
        r0 = i * TILE
        for t in range(2):
            cs = slice(t * HALF, (t + 1) * HALF)
            r1[c][t].wait_send()
            r1[c][t].wait_recv()
            out_ref[b, r0:r0 + TILE, cs] = (
                out_ref[b, r0:r0 + TILE, cs] + recvbuf[0, c, :, cs]
            )
            r2[c][t] = make_rdma(1, c, t)
            r2[c][t].start()

    def finish_r2(c):
        b, i = CHUNKS[c]
        r0 = i * TILE
        for t in range(2):
            cs = slice(t * HALF, (t + 1) * HALF)
            r2[c][t].wait_send()
            r2[c][t].wait_recv()
            out_ref[b, r0:r0 + TILE, cs] = (
                out_ref[b, r0:r0 + TILE, cs] + recvbuf[1, c, :, cs]
            )

    for c in range(C):
        compute_chunk(c)
        for t in range(2):
            r1[c][t] = make_rdma(0, c, t)
            r1[c][t].start()
        if c >= R1_DEPTH:
            finish_r1(c - R1_DEPTH)
        if c >= R2_DEPTH:
            finish_r2(c - R2_DEPTH)
    for c in range(C - R1_DEPTH, C):
        finish_r1(c)
    for c in range(C - R2_DEPTH, C):
        finish_r2(c)


def kernel(x, Wq, K_ext, V_ext, Wo):
    my = lax.axis_index("i")
    k_sl = lax.dynamic_slice_in_dim(K_ext, my * HQ_LOCAL, HQ_LOCAL, axis=2)
    v_sl = lax.dynamic_slice_in_dim(V_ext, my * HQ_LOCAL, HQ_LOCAL, axis=2)
    k_sl = k_sl.reshape(B, Skv, HQ_LOCAL * Dh).astype(jnp.bfloat16)
    v_sl = v_sl.reshape(B, Skv, HQ_LOCAL * Dh).astype(jnp.bfloat16)
    xb = x.astype(jnp.bfloat16)
    wq = (Wq * (SCALE * 1.4426950408889634)).astype(jnp.bfloat16)
    wo = Wo.astype(jnp.bfloat16)

    return pl.pallas_call(
        _body,
        out_shape=jax.ShapeDtypeStruct((B, Sq, D_MODEL), jnp.bfloat16),
        in_specs=[pl.BlockSpec(memory_space=pltpu.VMEM)] * 5,
        out_specs=pl.BlockSpec(memory_space=pltpu.VMEM),
        scratch_shapes=[
            pltpu.VMEM((B, Sq, HQ_LOCAL * Dh), jnp.bfloat16),
            pltpu.VMEM((2, C, TILE, D_MODEL), jnp.bfloat16),
            pltpu.SemaphoreType.DMA((2, C, 2)),
            pltpu.SemaphoreType.DMA((2, C, 2)),
        ],
        compiler_params=pltpu.CompilerParams(collective_id=0),
    )(xb, wq, k_sl, v_sl, wo)


# device time: 33518 ns/iter; 1.0056x vs baseline; 1.0056x over previous
import jax
import jax.numpy as jnp
from jax import lax
from jax.experimental import pallas as pl
from jax.experimental.pallas import tpu as pltpu

N_DEV = 4
B, Sq, Skv, Dh = 2, 512, 512, 64
HQ_LOCAL = 8
D_MODEL = 768
HALF = D_MODEL // 2
WINDOW = 128
SCALE = 0.125
TILE = 128
N_TILES = Sq // TILE
C = B * N_TILES
CHUNKS = [(b, i) for b in range(B) for i in range(N_TILES)]
R1_DEPTH = 2
R2_DEPTH = 5


def _band(i):
    return max(0, TILE * (i - 1)), min(Skv, TILE * (i + 2))


def _body(x_ref, wq_ref, k_ref, v_ref, wo_ref, out_ref,
          recvbuf, send_sems, recv_sems):
    my = lax.axis_index("i")
    p1 = my ^ 1
    p2 = 3 - my
    partners = ((p1, p2), (p2, p1))

    barrier_sem = pltpu.get_barrier_semaphore()
    for p in (p1, p2):
        pl.semaphore_signal(
            barrier_sem, inc=1,
            device_id=(p,), device_id_type=pl.DeviceIdType.MESH,
        )
    pl.semaphore_wait(barrier_sem, 2)

    def make_rdma(r, c, t):
        b, i = CHUNKS[c]
        r0 = i * TILE
        cs = slice(t * HALF, (t + 1) * HALF)
        return pltpu.make_async_remote_copy(
            src_ref=out_ref.at[b, r0:r0 + TILE, cs],
            dst_ref=recvbuf.at[r, c, :, cs],
            send_sem=send_sems.at[r, c, t],
            recv_sem=recv_sems.at[r, c, t],
            device_id=(partners[r][t],),
            device_id_type=pl.DeviceIdType.MESH,
        )

    def compute_chunk(c):
        b, i = CHUNKS[c]
        r0 = i * TILE
        xq = x_ref[b, r0:r0 + TILE, :]
        q_b = jnp.dot(
            xq, wq_ref[...], preferred_element_type=jnp.float32
        ).astype(jnp.bfloat16)
        j0, j1 = _band(i)
        colw = j1 - j0
        ri = lax.broadcasted_iota(jnp.int32, (TILE, colw), 0) + r0
        ci = lax.broadcasted_iota(jnp.int32, (TILE, colw), 1) + j0
        mask = jnp.abs(ri - ci) <= WINDOW
        ctx_cols = []
        for h in range(HQ_LOCAL):
            sl = slice(h * Dh, (h + 1) * Dh)
            q = q_b[:, sl]
            k = k_ref[b, j0:j1, sl]
            s = lax.dot_general(
                q, k, (((1,), (1,)), ((), ())),
                preferred_element_type=jnp.float32,
            )
            w = jnp.exp2(jnp.where(mask, s, -1e9))
            denom = 1.0 / jnp.sum(w, axis=1, keepdims=True)
            v = v_ref[b, j0:j1, sl]
            ctx_h = lax.dot_general(
                w.astype(jnp.bfloat16), v, (((1,), (0,)), ((), ())),
                preferred_element_type=jnp.float32,
            ) * denom
            ctx_cols.append(ctx_h.astype(jnp.bfloat16))
        ctx = jnp.concatenate(ctx_cols, axis=1)
        part = jnp.dot(
            ctx, wo_ref[...], preferred_element_type=jnp.float32
        )
        out_ref[b, r0:r0 + TILE, :] = part.astype(jnp.bfloat16)

    r1 = [[None, None] for _ in range(C)]
    r2 = [[None, None] for _ in range(C)]

    def finish_r1(c):
        b, i = CHUNKS[c]
        r0 = i * TILE
        for t in range(2):
            cs = slice(t * HALF, (t + 1) * HALF)
            r1[c][t].wait_send()
            r1[c][t].wait_recv()
            out_ref[b, r0:r0 + TILE, cs] = (
                out_ref[b, r0:r0 + TILE, cs] + recvbuf[0, c, :, cs]
            )
            r2[c][t] = make_rdma(1, c, t)
            r2[c][t].start()

    def finish_r2(c):
        b, i = CHUNKS[c]
        r0 = i * TILE
        for t in range(2):
            cs = slice(t * HALF, (t + 1) * HALF)
            r2[c][t].wait_send()
            r2[c][t].wait_recv()
            out_ref[b, r0:r0 + TILE, cs] = (
                out_ref[b, r0:r0 + TILE, cs] + recvbuf[1, c, :, cs]
            )

    for c in range(C):
        compute_chunk(c)
        for t in range(2):
            r1[c][t] = make_rdma(0, c, t)
            r1[c][t].start()
        if c >= R1_DEPTH:
            finish_r1(c - R1_DEPTH)
        if c >= R2_DEPTH:
            finish_r2(c - R2_DEPTH)
    for c in range(C - R1_DEPTH, C):
        finish_r1(c)
    for c in range(C - R2_DEPTH, C):
        finish_r2(c)


def kernel(x, Wq, K_ext, V_ext, Wo):
    my = lax.axis_index("i")
    k_sl = lax.dynamic_slice_in_dim(K_ext, my * HQ_LOCAL, HQ_LOCAL, axis=2)
    v_sl = lax.dynamic_slice_in_dim(V_ext, my * HQ_LOCAL, HQ_LOCAL, axis=2)
    k_sl = k_sl.reshape(B, Skv, HQ_LOCAL * Dh).astype(jnp.bfloat16)
    v_sl = v_sl.reshape(B, Skv, HQ_LOCAL * Dh).astype(jnp.bfloat16)
    xb = x.astype(jnp.bfloat16)
    wq = (Wq * (SCALE * 1.4426950408889634)).astype(jnp.bfloat16)
    wo = Wo.astype(jnp.bfloat16)

    return pl.pallas_call(
        _body,
        out_shape=jax.ShapeDtypeStruct((B, Sq, D_MODEL), jnp.bfloat16),
        in_specs=[pl.BlockSpec(memory_space=pltpu.VMEM)] * 5,
        out_specs=pl.BlockSpec(memory_space=pltpu.VMEM),
        scratch_shapes=[
            pltpu.VMEM((2, C, TILE, D_MODEL), jnp.bfloat16),
            pltpu.SemaphoreType.DMA((2, C, 2)),
            pltpu.SemaphoreType.DMA((2, C, 2)),
        ],
        compiler_params=pltpu.CompilerParams(collective_id=0),
    )(xb, wq, k_sl, v_sl, wo)


# device time: 32466 ns/iter; 1.0382x vs baseline; 1.0324x over previous
import jax
import jax.numpy as jnp
from jax import lax
from jax.experimental import pallas as pl
from jax.experimental.pallas import tpu as pltpu

N_DEV = 4
B, Sq, Skv, Dh = 2, 512, 512, 64
HQ_LOCAL = 8
D_MODEL = 768
HALF = D_MODEL // 2
WINDOW = 128
SCALE = 0.125
TILE = 128
N_TILES = Sq // TILE
C = B * N_TILES
CHUNKS = [(b, i) for b in range(B) for i in range(N_TILES)]
R1_DEPTH = 2
R2_DEPTH = 5


def _band(i):
    return max(0, TILE * (i - 1)), min(Skv, TILE * (i + 2))


def _body(x_ref, wq_ref, k_ref, v_ref, wo_ref, out_ref,
          recvbuf, send_sems, recv_sems):
    my = lax.axis_index("i")
    p1 = my ^ 1
    p2 = 3 - my
    partners = ((p1, p2), (p2, p1))

    barrier_sem = pltpu.get_barrier_semaphore()
    for p in (p1, p2):
        pl.semaphore_signal(
            barrier_sem, inc=1,
            device_id=(p,), device_id_type=pl.DeviceIdType.MESH,
        )
    pl.semaphore_wait(barrier_sem, 2)

    def make_rdma(r, c, t):
        b, i = CHUNKS[c]
        r0 = i * TILE
        cs = slice(t * HALF, (t + 1) * HALF)
        return pltpu.make_async_remote_copy(
            src_ref=out_ref.at[b, r0:r0 + TILE, cs],
            dst_ref=recvbuf.at[r, c, :, cs],
            send_sem=send_sems.at[r, c, t],
            recv_sem=recv_sems.at[r, c, t],
            device_id=(partners[r][t],),
            device_id_type=pl.DeviceIdType.MESH,
        )

    def compute_chunk(c):
        b, i = CHUNKS[c]
        r0 = i * TILE
        xq = x_ref[b, r0:r0 + TILE, :]
        q_b = jnp.dot(
            xq, wq_ref[...], preferred_element_type=jnp.float32
        ).astype(jnp.bfloat16)
        j0, j1 = _band(i)
        colw = j1 - j0
        ri = lax.broadcasted_iota(jnp.int32, (TILE, colw), 0) + r0
        ci = lax.broadcasted_iota(jnp.int32, (TILE, colw), 1) + j0
        mask = jnp.abs(ri - ci) <= WINDOW
        ctx_cols = []
        for h in range(HQ_LOCAL):
            sl = slice(h * Dh, (h + 1) * Dh)
            q = q_b[:, sl]
            k = k_ref[b, j0:j1, sl]
            s = lax.dot_general(
                q, k, (((1,), (1,)), ((), ())),
                preferred_element_type=jnp.float32,
            ).astype(jnp.bfloat16)
            w = jnp.exp2(jnp.where(mask, s, jnp.bfloat16(-1e9)))
            denom = 1.0 / jnp.sum(w, axis=1, keepdims=True,
                                  dtype=jnp.float32)
            v = v_ref[b, j0:j1, sl]
            ctx_h = lax.dot_general(
                w, v, (((1,), (0,)), ((), ())),
                preferred_element_type=jnp.float32,
            ) * denom
            ctx_cols.append(ctx_h.astype(jnp.bfloat16))
        ctx = jnp.concatenate(ctx_cols, axis=1)
        part = jnp.dot(
            ctx, wo_ref[...], preferred_element_type=jnp.float32
        )
        out_ref[b, r0:r0 + TILE, :] = part.astype(jnp.bfloat16)

    r1 = [[None, None] for _ in range(C)]
    r2 = [[None, None] for _ in range(C)]

    def finish_r1(c):
        b, i = CHUNKS[c]
        r0 = i * TILE
        for t in range(2):
            cs = slice(t * HALF, (t + 1) * HALF)
            r1[c][t].wait_send()
            r1[c][t].wait_recv()
            out_ref[b, r0:r0 + TILE, cs] = (
                out_ref[b, r0:r0 + TILE, cs] + recvbuf[0, c, :, cs]
            )
            r2[c][t] = make_rdma(1, c, t)
            r2[c][t].start()

    def finish_r2(c):
        b, i = CHUNKS[c]
        r0 = i * TILE
        for t in range(2):
            cs = slice(t * HALF, (t + 1) * HALF)
            r2[c][t].wait_send()
            r2[c][t].wait_recv()
            out_ref[b, r0:r0 + TILE, cs] = (
                out_ref[b, r0:r0 + TILE, cs] + recvbuf[1, c, :, cs]
            )

    for c in range(C):
        compute_chunk(c)
        for t in range(2):
            r1[c][t] = make_rdma(0, c, t)
            r1[c][t].start()
        if c >= R1_DEPTH:
            finish_r1(c - R1_DEPTH)
        if c >= R2_DEPTH:
            finish_r2(c - R2_DEPTH)
    for c in range(C - R1_DEPTH, C):
        finish_r1(c)
    for c in range(C - R2_DEPTH, C):
        finish_r2(c)


def kernel(x, Wq, K_ext, V_ext, Wo):
    my = lax.axis_index("i")
    k_sl = lax.dynamic_slice_in_dim(K_ext, my * HQ_LOCAL, HQ_LOCAL, axis=2)
    v_sl = lax.dynamic_slice_in_dim(V_ext, my * HQ_LOCAL, HQ_LOCAL, axis=2)
    k_sl = k_sl.reshape(B, Skv, HQ_LOCAL * Dh).astype(jnp.bfloat16)
    v_sl = v_sl.reshape(B, Skv, HQ_LOCAL * Dh).astype(jnp.bfloat16)
    xb = x.astype(jnp.bfloat16)
    wq = (Wq * (SCALE * 1.4426950408889634)).astype(jnp.bfloat16)
    wo = Wo.astype(jnp.bfloat16)

    return pl.pallas_call(
        _body,
        out_shape=jax.ShapeDtypeStruct((B, Sq, D_MODEL), jnp.bfloat16),
        in_specs=[pl.BlockSpec(memory_space=pltpu.VMEM)] * 5,
        out_specs=pl.BlockSpec(memory_space=pltpu.VMEM),
        scratch_shapes=[
            pltpu.VMEM((2, C, TILE, D_MODEL), jnp.bfloat16),
            pltpu.SemaphoreType.DMA((2, C, 2)),
            pltpu.SemaphoreType.DMA((2, C, 2)),
        ],
        compiler_params=pltpu.CompilerParams(collective_id=0),
    )(xb, wq, k_sl, v_sl, wo)
